# baseline (speedup 1.0000x reference)
import functools

import jax
import jax.numpy as jnp
from jax import lax
from jax.experimental import pallas as pl
from jax.experimental.pallas import tpu as pltpu
from jax.experimental.pallas import tpu_sc as plsc

NUM_KP = 1024
NMS_RADIUS = 4
MIN_SCORE = 0.01
MARGIN = 16
XF_MEAN = 0.485
XF_STD = 0.229

B = 4
H = W = 512
HD = WD = 64
NPATCH = HD * WD
SLABS = 8
SLAB_ROWS = H // SLABS
SLAB_ELEMS = SLAB_ROWS * W
NW = 32
CAP = 512
CCAP = SLABS * CAP
PAD_IDX = 1 << 30
KP_PER_W = B * NUM_KP // NW


def _nms_body(heat_ref, out_ref, flag_ref):
    x = heat_ref[0]
    ri = lax.broadcasted_iota(jnp.int32, (H, W), 0)
    ci = lax.broadcasted_iota(jnp.int32, (H, W), 1)
    inb = (ri >= MARGIN) & (ri < H - MARGIN) & (ci >= MARGIN) & (ci < W - MARGIN)
    x = jnp.where(inb, x, 0.0)
    neg = jnp.float32(-jnp.inf)

    def shift(a, s, axis):
        if axis == 0:
            if s > 0:
                return jnp.concatenate(
                    [a[s:, :], jnp.full((s, W), neg, jnp.float32)], axis=0)
            return jnp.concatenate(
                [jnp.full((-s, W), neg, jnp.float32), a[:s, :]], axis=0)
        if s > 0:
            return jnp.concatenate(
                [a[:, s:], jnp.full((H, s), neg, jnp.float32)], axis=1)
        return jnp.concatenate(
            [jnp.full((H, -s), neg, jnp.float32), a[:, :s]], axis=1)

    mp = x
    for s in range(1, NMS_RADIUS + 1):
        mp = jnp.maximum(mp, shift(x, s, 1))
        mp = jnp.maximum(mp, shift(x, -s, 1))
    mpv = mp
    for s in range(1, NMS_RADIUS + 1):
        mpv = jnp.maximum(mpv, shift(mp, s, 0))
        mpv = jnp.maximum(mpv, shift(mp, -s, 0))
    xn = x * (x == mpv).astype(jnp.float32)
    out_ref[0] = xn
    flag_ref[0] = jnp.max(xn.reshape(H, W // 16, 16), axis=2)


def _nms(heat):
    return pl.pallas_call(
        _nms_body,
        grid=(B,),
        in_specs=[pl.BlockSpec((1, H, W), lambda b: (b, 0, 0))],
        out_specs=[pl.BlockSpec((1, H, W), lambda b: (b, 0, 0)),
                   pl.BlockSpec((1, H, W // 16), lambda b: (b, 0, 0))],
        out_shape=[jax.ShapeDtypeStruct((B, H, W), jnp.float32),
                   jax.ShapeDtypeStruct((B, H, W // 16), jnp.float32)],
        compiler_params=pltpu.CompilerParams(
            dimension_semantics=("parallel",)),
    )(heat)


NGROUP = SLAB_ELEMS // 16


def _compact_body(heat_hbm, flag_hbm, val_hbm, idx_hbm,
                  slab_v, flag_v, wl_v, val_v, idx_v):
    c = lax.axis_index("c")
    s = lax.axis_index("s")
    wid = c * 16 + s
    pltpu.sync_copy(heat_hbm.at[wid], slab_v)
    pltpu.sync_copy(flag_hbm.at[wid], flag_v)
    base = (wid % SLABS) * SLAB_ELEMS

    zeros16 = jnp.zeros((16,), jnp.float32)
    pad16 = jnp.full((16,), PAD_IDX, jnp.int32)

    def init_body(i, carry):
        val_v[pl.ds(i * 16, 16)] = zeros16
        idx_v[pl.ds(i * 16, 16)] = pad16
        return carry

    lax.fori_loop(0, (CAP + 16) // 16, init_body, 0)

    lanes = lax.iota(jnp.int32, 16)

    def p1(i, wcnt):
        f = flag_v[pl.ds(i * 16, 16)]
        m = f > 0.0
        gidx = i * 16 + lanes
        plsc.store_compressed(wl_v.at[pl.ds(wcnt, 16)], gidx, mask=m)
        return wcnt + plsc.all_reduce_population_count(m)[0]

    wcnt = lax.fori_loop(0, NGROUP // 16, p1, jnp.int32(0))

    def p2(w, cnt):
        g = wl_v[pl.ds(w, 16)][0]
        v = slab_v[pl.ds(g * 16, 16)]
        m = v > 0.0
        idxv = (base + g * 16) + lanes
        cl = jnp.minimum(cnt, CAP)
        plsc.store_compressed(val_v.at[pl.ds(cl, 16)], v, mask=m)
        plsc.store_compressed(idx_v.at[pl.ds(cl, 16)], idxv, mask=m)
        return cnt + plsc.all_reduce_population_count(m)[0]

    lax.fori_loop(0, wcnt, p2, jnp.int32(0))
    pltpu.sync_copy(val_v.at[pl.ds(0, CAP)], val_hbm.at[wid])
    pltpu.sync_copy(idx_v.at[pl.ds(0, CAP)], idx_hbm.at[wid])


def _compact(heat_nms, flags):
    mesh = plsc.VectorSubcoreMesh(core_axis_name="c", subcore_axis_name="s")
    f = pl.kernel(
        _compact_body,
        out_type=[
            jax.ShapeDtypeStruct((NW, CAP), jnp.float32),
            jax.ShapeDtypeStruct((NW, CAP), jnp.int32),
        ],
        mesh=mesh,
        compiler_params=pltpu.CompilerParams(needs_layout_passes=False),
        scratch_types=[
            pltpu.VMEM((SLAB_ELEMS,), jnp.float32),
            pltpu.VMEM((NGROUP,), jnp.float32),
            pltpu.VMEM((NGROUP + 16,), jnp.int32),
            pltpu.VMEM((CAP + 16,), jnp.float32),
            pltpu.VMEM((CAP + 16,), jnp.int32),
        ],
    )
    return f(heat_nms.reshape(NW, SLAB_ELEMS), flags.reshape(NW, NGROUP))


def _rank_body(vrow_ref, irow_ref, vcol_ref, icol_ref,
               kps_ref, sc_ref, sel_ref, valid_ref, rank_scr):
    vrow = vrow_ref[0]
    irow = irow_ref[0]

    njb = CCAP // 512

    def jblock(jb, carry):
        vj = vcol_ref[0, pl.ds(jb * 512, 512), :]
        ij = icol_ref[0, pl.ds(jb * 512, 512), :]
        beat = (vrow > vj) | ((vrow == vj) & (irow < ij))
        cnt = jnp.sum(beat.astype(jnp.float32), axis=1, keepdims=True)
        rank_scr[pl.ds(jb * 512, 512), :] = cnt
        return carry

    lax.fori_loop(0, njb, jblock, 0)

    rank_col = rank_scr[:, :].astype(jnp.int32)
    vcol = vcol_ref[0]
    sel = vcol > 0.0
    slot = lax.broadcasted_iota(jnp.int32, (CCAP, NUM_KP), 1)
    pt = ((rank_col == slot) & sel).astype(jnp.float32)
    payload = jnp.concatenate(
        [vrow, irow.astype(jnp.float32)], axis=0)
    res = jax.lax.dot(payload, pt,
                      precision=lax.Precision.HIGHEST,
                      preferred_element_type=jnp.float32)

    n_pos = jnp.sum(sel.astype(jnp.int32))
    slot1 = lax.broadcasted_iota(jnp.int32, (1, NUM_KP), 1)
    fill = slot1 >= n_pos
    sval = jnp.where(fill, 0.0, res[0:1, :])
    sidx = jnp.where(fill, slot1 - n_pos, res[1:2, :].astype(jnp.int32))

    xs = (sidx % W).astype(jnp.float32)
    ys = (sidx // W).astype(jnp.float32)
    kps_ref[0] = jnp.concatenate([xs, ys], axis=0)
    validf = (sval > MIN_SCORE).astype(jnp.float32)
    sc_ref[0] = sval * validf
    sel_ref[0] = sidx
    valid_ref[0] = validf


def _rank_select(cand_val, cand_idx):
    vrow = cand_val.reshape(B, 1, CCAP)
    irow = cand_idx.reshape(B, 1, CCAP)
    vcol = cand_val.reshape(B, CCAP, 1)
    icol = cand_idx.reshape(B, CCAP, 1)
    return pl.pallas_call(
        _rank_body,
        grid=(B,),
        in_specs=[
            pl.BlockSpec((1, 1, CCAP), lambda b: (b, 0, 0)),
            pl.BlockSpec((1, 1, CCAP), lambda b: (b, 0, 0)),
            pl.BlockSpec((1, CCAP, 1), lambda b: (b, 0, 0)),
            pl.BlockSpec((1, CCAP, 1), lambda b: (b, 0, 0)),
        ],
        out_specs=[
            pl.BlockSpec((1, 2, NUM_KP), lambda b: (b, 0, 0)),
            pl.BlockSpec((1, 1, NUM_KP), lambda b: (b, 0, 0)),
            pl.BlockSpec((1, 1, NUM_KP), lambda b: (b, 0, 0)),
            pl.BlockSpec((1, 1, NUM_KP), lambda b: (b, 0, 0)),
        ],
        out_shape=[
            jax.ShapeDtypeStruct((B, 2, NUM_KP), jnp.float32),
            jax.ShapeDtypeStruct((B, 1, NUM_KP), jnp.float32),
            jax.ShapeDtypeStruct((B, 1, NUM_KP), jnp.int32),
            jax.ShapeDtypeStruct((B, 1, NUM_KP), jnp.float32),
        ],
        scratch_shapes=[pltpu.VMEM((CCAP, 1), jnp.float32)],
        compiler_params=pltpu.CompilerParams(
            dimension_semantics=("parallel",)),
    )(vrow, irow, vcol, icol)


def _dense_body(p_ref, wk_ref, wd_ref, prob_ref, desc_ref):
    p = p_ref[0]
    xn = (p - XF_MEAN) / XF_STD
    logits = jax.lax.dot(xn, wk_ref[...],
                         precision=lax.Precision.DEFAULT,
                         preferred_element_type=jnp.float32)
    m = jnp.max(logits, axis=1, keepdims=True)
    e = jnp.exp(logits - m)
    s = jnp.sum(e, axis=1, keepdims=True)
    prob_ref[0] = (e / s)[:, :64]
    xc = jnp.clip(p, 0.0, 1.0)
    desc_ref[0] = jax.lax.dot(xc, wd_ref[...],
                              precision=lax.Precision.DEFAULT,
                              preferred_element_type=jnp.float32)


def _dense(patches, wkt, wdt):
    return pl.pallas_call(
        _dense_body,
        grid=(B,),
        in_specs=[
            pl.BlockSpec((1, NPATCH, 64), lambda b: (b, 0, 0)),
            pl.BlockSpec((64, 65), lambda b: (0, 0)),
            pl.BlockSpec((64, 256), lambda b: (0, 0)),
        ],
        out_specs=[
            pl.BlockSpec((1, NPATCH, 64), lambda b: (b, 0, 0)),
            pl.BlockSpec((1, NPATCH, 256), lambda b: (b, 0, 0)),
        ],
        out_shape=[
            jax.ShapeDtypeStruct((B, NPATCH, 64), jnp.float32),
            jax.ShapeDtypeStruct((B, NPATCH, 256), jnp.float32),
        ],
        compiler_params=pltpu.CompilerParams(
            dimension_semantics=("parallel",)),
    )(patches, wkt, wdt)


def _bilerp_body(desc_hbm, sel_hbm, out_hbm,
                 idx_v, abuf, bbuf, cbuf, dbuf, obuf, sem):
    c = lax.axis_index("c")
    s = lax.axis_index("s")
    wid = c * 16 + s
    kp0 = wid * KP_PER_W
    img_base = (wid // (NUM_KP // KP_PER_W)) * NPATCH
    pltpu.sync_copy(sel_hbm.at[pl.ds(kp0, KP_PER_W)], idx_v)

    lanes = lax.iota(jnp.int32, 16)

    def batch(b8, carry):
        idx16 = idx_v[pl.ds(b8 * 16, 16)]
        xs = (idx16 % W).astype(jnp.float32)
        ys = (idx16 // W).astype(jnp.float32)
        gx = 2.0 * xs / (W - 1) - 1.0
        gy = 2.0 * ys / (H - 1) - 1.0
        ix = ((gx + 1.0) * WD - 1.0) / 2.0
        iy = ((gy + 1.0) * HD - 1.0) / 2.0
        x0i = ix.astype(jnp.int32)
        y0i = iy.astype(jnp.int32)
        x0f = x0i.astype(jnp.float32)
        y0f = y0i.astype(jnp.float32)
        x1f = x0f + 1.0
        y1f = y0f + 1.0
        wa = (x1f - ix) * (y1f - iy)
        wb = (ix - x0f) * (y1f - iy)
        wc = (x1f - ix) * (iy - y0f)
        wd = (ix - x0f) * (iy - y0f)
        x0c = jnp.clip(x0i, 0, WD - 1)
        x1c = jnp.clip(x0i + 1, 0, WD - 1)
        y0c = jnp.clip(y0i, 0, HD - 1)
        y1c = jnp.clip(y0i + 1, 0, HD - 1)
        r00 = img_base + y0c * WD + x0c
        r01 = img_base + y0c * WD + x1c
        r10 = img_base + y1c * WD + x0c
        r11 = img_base + y1c * WD + x1c
        ca = pltpu.async_copy(desc_hbm.at[r00], abuf, sem)
        cb = pltpu.async_copy(desc_hbm.at[r01], bbuf, sem)
        cc = pltpu.async_copy(desc_hbm.at[r10], cbuf, sem)
        cd = pltpu.async_copy(desc_hbm.at[r11], dbuf, sem)
        ca.wait()
        cb.wait()
        cc.wait()
        cd.wait()

        def chan(ch, carry2):
            chv = jnp.full((16,), 0, jnp.int32) + ch
            va = plsc.load_gather(abuf, [lanes, chv])
            vb = plsc.load_gather(bbuf, [lanes, chv])
            vc = plsc.load_gather(cbuf, [lanes, chv])
            vd = plsc.load_gather(dbuf, [lanes, chv])
            acc = wa * va + wb * vb + wc * vc + wd * vd
            plsc.store_scatter(obuf, [lanes, chv], acc)
            return carry2

        lax.fori_loop(0, 256, chan, 0)
        pltpu.sync_copy(obuf, out_hbm.at[pl.ds(kp0 + b8 * 16, 16)])
        return carry

    lax.fori_loop(0, KP_PER_W // 16, batch, 0)


def _bilerp(desc_flat, sel_idx):
    mesh = plsc.VectorSubcoreMesh(core_axis_name="c", subcore_axis_name="s")
    f = pl.kernel(
        _bilerp_body,
        out_type=[jax.ShapeDtypeStruct((B * NUM_KP, 256), jnp.float32)],
        mesh=mesh,
        compiler_params=pltpu.CompilerParams(needs_layout_passes=False),
        scratch_types=[
            pltpu.VMEM((KP_PER_W,), jnp.int32),
            pltpu.VMEM((16, 256), jnp.float32),
            pltpu.VMEM((16, 256), jnp.float32),
            pltpu.VMEM((16, 256), jnp.float32),
            pltpu.VMEM((16, 256), jnp.float32),
            pltpu.VMEM((16, 256), jnp.float32),
            pltpu.SemaphoreType.DMA,
        ],
    )
    (out,) = f(desc_flat, sel_idx)
    return out


def _norm_body(d_ref, valid_ref, out_ref):
    d = d_ref[0]
    v = valid_ref[0]
    ss = jnp.sum(d * d, axis=1, keepdims=True)
    out_ref[0] = d / (jnp.sqrt(ss) + 1e-8) * v


def _normalize(desc_raw, valid):
    return pl.pallas_call(
        _norm_body,
        grid=(B,),
        in_specs=[
            pl.BlockSpec((1, NUM_KP, 256), lambda b: (b, 0, 0)),
            pl.BlockSpec((1, NUM_KP, 1), lambda b: (b, 0, 0)),
        ],
        out_specs=pl.BlockSpec((1, NUM_KP, 256), lambda b: (b, 0, 0)),
        out_shape=jax.ShapeDtypeStruct((B, NUM_KP, 256), jnp.float32),
        compiler_params=pltpu.CompilerParams(
            dimension_semantics=("parallel",)),
    )(desc_raw, valid)


def kernel(image, W_kp, W_desc):
    patches = image.reshape(B, HD, 8, WD, 8).transpose(0, 1, 3, 2, 4)
    patches = patches.reshape(B, NPATCH, 64)
    wkt = W_kp.reshape(65, 64).T
    wdt = W_desc.reshape(256, 64).T
    prob, desc_flat = _dense(patches, wkt, wdt)

    heat = prob.reshape(B, HD, WD, 8, 8).transpose(0, 1, 3, 2, 4)
    heat = heat.reshape(B, H, W)

    heat_nms, flags = _nms(heat)

    cand_val, cand_idx = _compact(heat_nms, flags)
    cand_val = cand_val.reshape(B, CCAP)
    cand_idx = cand_idx.reshape(B, CCAP)

    kps2, sc3, sel3, valid3 = _rank_select(cand_val, cand_idx)
    kps = kps2.transpose(0, 2, 1)
    sc = sc3.reshape(B, NUM_KP)

    desc_raw = _bilerp(desc_flat.reshape(B * NPATCH, 256),
                       sel3.reshape(B * NUM_KP))
    desc = _normalize(desc_raw.reshape(B, NUM_KP, 256),
                      valid3.reshape(B, NUM_KP, 1))
    return kps, sc, desc


# speedup vs baseline: 1.2055x; 1.2055x over previous
import functools

import jax
import jax.numpy as jnp
from jax import lax
from jax.experimental import pallas as pl
from jax.experimental.pallas import tpu as pltpu
from jax.experimental.pallas import tpu_sc as plsc

NUM_KP = 1024
NMS_RADIUS = 4
MIN_SCORE = 0.01
MARGIN = 16
XF_MEAN = 0.485
XF_STD = 0.229

B = 4
H = W = 512
HD = WD = 64
NPATCH = HD * WD
SLABS = 8
SLAB_ROWS = H // SLABS
SLAB_ELEMS = SLAB_ROWS * W
NW = 32
CAP = 512
CCAP = SLABS * CAP
PAD_IDX = 1 << 30
KP_PER_W = B * NUM_KP // NW


def _nms_body(heat_ref, out_ref, flag_ref):
    x = heat_ref[0]
    ri = lax.broadcasted_iota(jnp.int32, (H, W), 0)
    ci = lax.broadcasted_iota(jnp.int32, (H, W), 1)
    inb = (ri >= MARGIN) & (ri < H - MARGIN) & (ci >= MARGIN) & (ci < W - MARGIN)
    x = jnp.where(inb, x, 0.0)
    neg = jnp.float32(-jnp.inf)

    def shift(a, s, axis):
        if axis == 0:
            if s > 0:
                return jnp.concatenate(
                    [a[s:, :], jnp.full((s, W), neg, jnp.float32)], axis=0)
            return jnp.concatenate(
                [jnp.full((-s, W), neg, jnp.float32), a[:s, :]], axis=0)
        if s > 0:
            return jnp.concatenate(
                [a[:, s:], jnp.full((H, s), neg, jnp.float32)], axis=1)
        return jnp.concatenate(
            [jnp.full((H, -s), neg, jnp.float32), a[:, :s]], axis=1)

    mp = x
    for s in range(1, NMS_RADIUS + 1):
        mp = jnp.maximum(mp, shift(x, s, 1))
        mp = jnp.maximum(mp, shift(x, -s, 1))
    mpv = mp
    for s in range(1, NMS_RADIUS + 1):
        mpv = jnp.maximum(mpv, shift(mp, s, 0))
        mpv = jnp.maximum(mpv, shift(mp, -s, 0))
    xn = x * (x == mpv).astype(jnp.float32)
    out_ref[0] = xn
    flag_ref[0] = jnp.max(xn.reshape(H, W // 16, 16), axis=2)


def _nms(heat):
    return pl.pallas_call(
        _nms_body,
        grid=(B,),
        in_specs=[pl.BlockSpec((1, H, W), lambda b: (b, 0, 0))],
        out_specs=[pl.BlockSpec((1, H, W), lambda b: (b, 0, 0)),
                   pl.BlockSpec((1, H, W // 16), lambda b: (b, 0, 0))],
        out_shape=[jax.ShapeDtypeStruct((B, H, W), jnp.float32),
                   jax.ShapeDtypeStruct((B, H, W // 16), jnp.float32)],
        compiler_params=pltpu.CompilerParams(
            dimension_semantics=("parallel",)),
    )(heat)


NGROUP = SLAB_ELEMS // 16


def _compact_body(heat_hbm, flag_hbm, val_hbm, idx_hbm,
                  slab_v, flag_v, wl_v, val_v, idx_v):
    c = lax.axis_index("c")
    s = lax.axis_index("s")
    wid = c * 16 + s
    pltpu.sync_copy(heat_hbm.at[wid], slab_v)
    pltpu.sync_copy(flag_hbm.at[wid], flag_v)
    base = (wid % SLABS) * SLAB_ELEMS

    zeros16 = jnp.zeros((16,), jnp.float32)
    pad16 = jnp.full((16,), PAD_IDX, jnp.int32)

    def init_body(i, carry):
        val_v[pl.ds(i * 16, 16)] = zeros16
        idx_v[pl.ds(i * 16, 16)] = pad16
        return carry

    lax.fori_loop(0, (CAP + 16) // 16, init_body, 0)

    lanes = lax.iota(jnp.int32, 16)

    def p1(i, wcnt):
        f = flag_v[pl.ds(i * 16, 16)]
        m = f > 0.0
        gidx = i * 16 + lanes
        plsc.store_compressed(wl_v.at[pl.ds(wcnt, 16)], gidx, mask=m)
        return wcnt + plsc.all_reduce_population_count(m)[0]

    wcnt = lax.fori_loop(0, NGROUP // 16, p1, jnp.int32(0))

    def p2(w, cnt):
        g = wl_v[pl.ds(w, 16)][0]
        v = slab_v[pl.ds(g * 16, 16)]
        m = v > 0.0
        idxv = (base + g * 16) + lanes
        cl = jnp.minimum(cnt, CAP)
        plsc.store_compressed(val_v.at[pl.ds(cl, 16)], v, mask=m)
        plsc.store_compressed(idx_v.at[pl.ds(cl, 16)], idxv, mask=m)
        return cnt + plsc.all_reduce_population_count(m)[0]

    lax.fori_loop(0, wcnt, p2, jnp.int32(0))
    pltpu.sync_copy(val_v.at[pl.ds(0, CAP)], val_hbm.at[wid])
    pltpu.sync_copy(idx_v.at[pl.ds(0, CAP)], idx_hbm.at[wid])


def _compact(heat_nms, flags):
    mesh = plsc.VectorSubcoreMesh(core_axis_name="c", subcore_axis_name="s")
    f = pl.kernel(
        _compact_body,
        out_type=[
            jax.ShapeDtypeStruct((NW, CAP), jnp.float32),
            jax.ShapeDtypeStruct((NW, CAP), jnp.int32),
        ],
        mesh=mesh,
        compiler_params=pltpu.CompilerParams(needs_layout_passes=False),
        scratch_types=[
            pltpu.VMEM((SLAB_ELEMS,), jnp.float32),
            pltpu.VMEM((NGROUP,), jnp.float32),
            pltpu.VMEM((NGROUP + 16,), jnp.int32),
            pltpu.VMEM((CAP + 16,), jnp.float32),
            pltpu.VMEM((CAP + 16,), jnp.int32),
        ],
    )
    return f(heat_nms.reshape(NW, SLAB_ELEMS), flags.reshape(NW, NGROUP))


def _rank_body(vrow_ref, irow_ref, vcol_ref, icol_ref,
               kps_ref, sc_ref, sel_ref, valid_ref, rank_scr):
    vrow = vrow_ref[0]
    irow = irow_ref[0]

    njb = CCAP // 512

    def jblock(jb, carry):
        vj = vcol_ref[0, pl.ds(jb * 512, 512), :]
        ij = icol_ref[0, pl.ds(jb * 512, 512), :]
        beat = (vrow > vj) | ((vrow == vj) & (irow < ij))
        cnt = jnp.sum(beat.astype(jnp.float32), axis=1, keepdims=True)
        rank_scr[pl.ds(jb * 512, 512), :] = cnt
        return carry

    lax.fori_loop(0, njb, jblock, 0)

    rank_col = rank_scr[:, :].astype(jnp.int32)
    vcol = vcol_ref[0]
    sel = vcol > 0.0
    slot = lax.broadcasted_iota(jnp.int32, (CCAP, NUM_KP), 1)
    pt = ((rank_col == slot) & sel).astype(jnp.float32)
    payload = jnp.concatenate(
        [vrow, irow.astype(jnp.float32)], axis=0)
    res = jax.lax.dot(payload, pt,
                      precision=lax.Precision.HIGHEST,
                      preferred_element_type=jnp.float32)

    n_pos = jnp.sum(sel.astype(jnp.int32))
    slot1 = lax.broadcasted_iota(jnp.int32, (1, NUM_KP), 1)
    fill = slot1 >= n_pos
    sval = jnp.where(fill, 0.0, res[0:1, :])
    sidx = jnp.where(fill, slot1 - n_pos, res[1:2, :].astype(jnp.int32))

    xs = (sidx % W).astype(jnp.float32)
    ys = (sidx // W).astype(jnp.float32)
    kps_ref[0] = jnp.concatenate([xs, ys], axis=0)
    validf = (sval > MIN_SCORE).astype(jnp.float32)
    sc_ref[0] = sval * validf
    sel_ref[0] = sidx
    valid_ref[0] = validf


def _rank_select(cand_val, cand_idx):
    vrow = cand_val.reshape(B, 1, CCAP)
    irow = cand_idx.reshape(B, 1, CCAP)
    vcol = cand_val.reshape(B, CCAP, 1)
    icol = cand_idx.reshape(B, CCAP, 1)
    return pl.pallas_call(
        _rank_body,
        grid=(B,),
        in_specs=[
            pl.BlockSpec((1, 1, CCAP), lambda b: (b, 0, 0)),
            pl.BlockSpec((1, 1, CCAP), lambda b: (b, 0, 0)),
            pl.BlockSpec((1, CCAP, 1), lambda b: (b, 0, 0)),
            pl.BlockSpec((1, CCAP, 1), lambda b: (b, 0, 0)),
        ],
        out_specs=[
            pl.BlockSpec((1, 2, NUM_KP), lambda b: (b, 0, 0)),
            pl.BlockSpec((1, 1, NUM_KP), lambda b: (b, 0, 0)),
            pl.BlockSpec((1, 1, NUM_KP), lambda b: (b, 0, 0)),
            pl.BlockSpec((1, 1, NUM_KP), lambda b: (b, 0, 0)),
        ],
        out_shape=[
            jax.ShapeDtypeStruct((B, 2, NUM_KP), jnp.float32),
            jax.ShapeDtypeStruct((B, 1, NUM_KP), jnp.float32),
            jax.ShapeDtypeStruct((B, 1, NUM_KP), jnp.int32),
            jax.ShapeDtypeStruct((B, 1, NUM_KP), jnp.float32),
        ],
        scratch_shapes=[pltpu.VMEM((CCAP, 1), jnp.float32)],
        compiler_params=pltpu.CompilerParams(
            dimension_semantics=("parallel",)),
    )(vrow, irow, vcol, icol)


def _dense_body(p_ref, wk_ref, wd_ref, prob_ref, desc_ref):
    p = p_ref[0]
    xn = (p - XF_MEAN) / XF_STD
    logits = jax.lax.dot(xn, wk_ref[...],
                         precision=lax.Precision.DEFAULT,
                         preferred_element_type=jnp.float32)
    m = jnp.max(logits, axis=1, keepdims=True)
    e = jnp.exp(logits - m)
    s = jnp.sum(e, axis=1, keepdims=True)
    prob_ref[0] = (e / s)[:, :64]
    xc = jnp.clip(p, 0.0, 1.0)
    desc_ref[0] = jax.lax.dot(xc, wd_ref[...],
                              precision=lax.Precision.DEFAULT,
                              preferred_element_type=jnp.float32)


def _dense(patches, wkt, wdt):
    return pl.pallas_call(
        _dense_body,
        grid=(B,),
        in_specs=[
            pl.BlockSpec((1, NPATCH, 64), lambda b: (b, 0, 0)),
            pl.BlockSpec((64, 65), lambda b: (0, 0)),
            pl.BlockSpec((64, 256), lambda b: (0, 0)),
        ],
        out_specs=[
            pl.BlockSpec((1, NPATCH, 64), lambda b: (b, 0, 0)),
            pl.BlockSpec((1, NPATCH, 256), lambda b: (b, 0, 0)),
        ],
        out_shape=[
            jax.ShapeDtypeStruct((B, NPATCH, 64), jnp.float32),
            jax.ShapeDtypeStruct((B, NPATCH, 256), jnp.float32),
        ],
        compiler_params=pltpu.CompilerParams(
            dimension_semantics=("parallel",)),
    )(patches, wkt, wdt)


def _bilerp_body(desc_hbm, sel_hbm, out_hbm,
                 idx_v, abuf, bbuf, cbuf, dbuf, obuf, w_v, sem):
    c = lax.axis_index("c")
    s = lax.axis_index("s")
    wid = c * 16 + s
    kp0 = wid * KP_PER_W
    img_base = (wid // (NUM_KP // KP_PER_W)) * NPATCH
    pltpu.sync_copy(sel_hbm.at[pl.ds(kp0, KP_PER_W)], idx_v)

    lanes = lax.iota(jnp.int32, 16)

    def batch(b8, carry):
        idx16 = idx_v[pl.ds(b8 * 16, 16)]
        xs = (idx16 % W).astype(jnp.float32)
        ys = (idx16 // W).astype(jnp.float32)
        gx = 2.0 * xs / (W - 1) - 1.0
        gy = 2.0 * ys / (H - 1) - 1.0
        ix = ((gx + 1.0) * WD - 1.0) / 2.0
        iy = ((gy + 1.0) * HD - 1.0) / 2.0
        x0i = ix.astype(jnp.int32)
        y0i = iy.astype(jnp.int32)
        x0f = x0i.astype(jnp.float32)
        y0f = y0i.astype(jnp.float32)
        x1f = x0f + 1.0
        y1f = y0f + 1.0
        wa = (x1f - ix) * (y1f - iy)
        wb = (ix - x0f) * (y1f - iy)
        wc = (x1f - ix) * (iy - y0f)
        wd = (ix - x0f) * (iy - y0f)
        x0c = jnp.clip(x0i, 0, WD - 1)
        x1c = jnp.clip(x0i + 1, 0, WD - 1)
        y0c = jnp.clip(y0i, 0, HD - 1)
        y1c = jnp.clip(y0i + 1, 0, HD - 1)
        r00 = img_base + y0c * WD + x0c
        r01 = img_base + y0c * WD + x1c
        r10 = img_base + y1c * WD + x0c
        r11 = img_base + y1c * WD + x1c
        ca = pltpu.async_copy(desc_hbm.at[r00], abuf, sem)
        cb = pltpu.async_copy(desc_hbm.at[r01], bbuf, sem)
        cc = pltpu.async_copy(desc_hbm.at[r10], cbuf, sem)
        cd = pltpu.async_copy(desc_hbm.at[r11], dbuf, sem)
        w_v[pl.ds(0, 16)] = wa
        w_v[pl.ds(16, 16)] = wb
        w_v[pl.ds(32, 16)] = wc
        w_v[pl.ds(48, 16)] = wd
        ca.wait()
        cb.wait()
        cc.wait()
        cd.wait()

        def kp_loop(i, carry2):
            iv = jnp.full((16,), 0, jnp.int32) + i
            wa_b = plsc.load_gather(w_v, [iv])
            wb_b = plsc.load_gather(w_v, [iv + 16])
            wc_b = plsc.load_gather(w_v, [iv + 32])
            wd_b = plsc.load_gather(w_v, [iv + 48])

            def chunk(ch, carry3):
                sl = pl.ds(ch * 16, 16)
                obuf[i, sl] = (wa_b * abuf[i, sl] + wb_b * bbuf[i, sl]
                               + wc_b * cbuf[i, sl] + wd_b * dbuf[i, sl])
                return carry3

            lax.fori_loop(0, 256 // 16, chunk, 0)
            return carry2

        lax.fori_loop(0, 16, kp_loop, 0)
        pltpu.sync_copy(obuf, out_hbm.at[pl.ds(kp0 + b8 * 16, 16)])
        return carry

    lax.fori_loop(0, KP_PER_W // 16, batch, 0)


def _bilerp(desc_flat, sel_idx):
    mesh = plsc.VectorSubcoreMesh(core_axis_name="c", subcore_axis_name="s")
    f = pl.kernel(
        _bilerp_body,
        out_type=[jax.ShapeDtypeStruct((B * NUM_KP, 256), jnp.float32)],
        mesh=mesh,
        compiler_params=pltpu.CompilerParams(needs_layout_passes=False),
        scratch_types=[
            pltpu.VMEM((KP_PER_W,), jnp.int32),
            pltpu.VMEM((16, 256), jnp.float32),
            pltpu.VMEM((16, 256), jnp.float32),
            pltpu.VMEM((16, 256), jnp.float32),
            pltpu.VMEM((16, 256), jnp.float32),
            pltpu.VMEM((16, 256), jnp.float32),
            pltpu.VMEM((64,), jnp.float32),
            pltpu.SemaphoreType.DMA,
        ],
    )
    (out,) = f(desc_flat, sel_idx)
    return out


def _norm_body(d_ref, valid_ref, out_ref):
    d = d_ref[0]
    v = valid_ref[0]
    ss = jnp.sum(d * d, axis=1, keepdims=True)
    out_ref[0] = d / (jnp.sqrt(ss) + 1e-8) * v


def _normalize(desc_raw, valid):
    return pl.pallas_call(
        _norm_body,
        grid=(B,),
        in_specs=[
            pl.BlockSpec((1, NUM_KP, 256), lambda b: (b, 0, 0)),
            pl.BlockSpec((1, NUM_KP, 1), lambda b: (b, 0, 0)),
        ],
        out_specs=pl.BlockSpec((1, NUM_KP, 256), lambda b: (b, 0, 0)),
        out_shape=jax.ShapeDtypeStruct((B, NUM_KP, 256), jnp.float32),
        compiler_params=pltpu.CompilerParams(
            dimension_semantics=("parallel",)),
    )(desc_raw, valid)


def kernel(image, W_kp, W_desc):
    patches = image.reshape(B, HD, 8, WD, 8).transpose(0, 1, 3, 2, 4)
    patches = patches.reshape(B, NPATCH, 64)
    wkt = W_kp.reshape(65, 64).T
    wdt = W_desc.reshape(256, 64).T
    prob, desc_flat = _dense(patches, wkt, wdt)

    heat = prob.reshape(B, HD, WD, 8, 8).transpose(0, 1, 3, 2, 4)
    heat = heat.reshape(B, H, W)

    heat_nms, flags = _nms(heat)

    cand_val, cand_idx = _compact(heat_nms, flags)
    cand_val = cand_val.reshape(B, CCAP)
    cand_idx = cand_idx.reshape(B, CCAP)

    kps2, sc3, sel3, valid3 = _rank_select(cand_val, cand_idx)
    kps = kps2.transpose(0, 2, 1)
    sc = sc3.reshape(B, NUM_KP)

    desc_raw = _bilerp(desc_flat.reshape(B * NPATCH, 256),
                       sel3.reshape(B * NUM_KP))
    desc = _normalize(desc_raw.reshape(B, NUM_KP, 256),
                      valid3.reshape(B, NUM_KP, 1))
    return kps, sc, desc
